# baseline (speedup 1.0000x reference)
import functools

import jax
import jax.numpy as jnp
from jax import lax
from jax.experimental import pallas as pl
from jax.experimental.pallas import tpu as pltpu
from jax.experimental.pallas import tpu_sc as plsc

_N = 32
_S = 4
_D = 32
_G = 32
_T0 = 28
_NT = 4
_LO = 26
_NR = 6
_NW = 32
_CH = 64


def _prep_body(tab_ref, ws_ref, wm_ref, bg_ref, src_ref, dst_ref, out_ref):
    t6 = tab_ref[_LO * _S:(_LO + _NR) * _S, :]
    ps = jnp.dot(t6, ws_ref[...], preferred_element_type=jnp.float32)
    pm = jnp.dot(t6, wm_ref[...], preferred_element_type=jnp.float32)

    e = src_ref.shape[1]
    scol = src_ref[...]
    dcol = dst_ref[...]
    kio = lax.broadcasted_iota(jnp.int32, (_NT, e), 0) + _T0
    ok = (jnp.broadcast_to(dcol, (_NT, e)) == kio).astype(jnp.float32)
    rio = lax.broadcasted_iota(jnp.int32, (_NR * _S, e), 0) // _S + _LO
    sel = (jnp.broadcast_to(scol, (_NR * _S, e)) == rio).astype(jnp.float32)
    cnt = lax.dot_general(sel, ok, (((1,), (1,)), ((), ())),
                          preferred_element_type=jnp.float32)
    deg = lax.dot_general(jnp.ones((1, e), jnp.float32), ok,
                          (((1,), (1,)), ((), ())),
                          preferred_element_type=jnp.float32)
    coeff = cnt / jnp.maximum(deg, 1.0)
    r24 = lax.broadcasted_iota(jnp.int32, (_NR * _S, _NT), 0) // _S + _LO
    k24 = lax.broadcasted_iota(jnp.int32, (_NR * _S, _NT), 1) + _T0
    delta = (r24 == k24).astype(jnp.float32)

    blocks = []
    for k in range(_NT):
        blocks.append(delta[:, k:k + 1] * ps + coeff[:, k:k + 1] * pm)
    m24 = jnp.concatenate(blocks, axis=1)

    ci = lax.broadcasted_iota(jnp.int32, (_S ** 3, _NR * _S), 0)
    ri = lax.broadcasted_iota(jnp.int32, (_S ** 3, _NR * _S), 1)
    node = ri // _S
    st = ri % _S
    sel_a = (((node == 0) & (st == ci // 16))
             | ((node == 1) & (st == (ci // 4) % 4))
             | ((node == 2) & (st == ci % 4))).astype(jnp.float32)
    sel_b = (((node == 3) & (st == ci // 16))
             | ((node == 4) & (st == (ci // 4) % 4))
             | ((node == 5) & (st == ci % 4))).astype(jnp.float32)
    ta = jnp.dot(sel_a, m24, preferred_element_type=jnp.float32)
    tb = jnp.dot(sel_b, m24, preferred_element_type=jnp.float32)
    btile = jnp.concatenate([bg_ref[...]] * _NT, axis=1)
    ta = ta + btile
    fi = lax.broadcasted_iota(jnp.int32, (_S ** 6, _S ** 3), 0)
    fj = lax.broadcasted_iota(jnp.int32, (_S ** 6, _S ** 3), 1)
    rep_a = (fi // (_S ** 3) == fj).astype(jnp.float32)
    rep_b = (fi % (_S ** 3) == fj).astype(jnp.float32)
    full = (jnp.dot(rep_a, ta, preferred_element_type=jnp.float32)
            + jnp.dot(rep_b, tb, preferred_element_type=jnp.float32))
    out_ref[...] = jnp.maximum(full, full * 0.01)


def _prep(tab, ws, wm, bg, src2, dst2):
    return pl.pallas_call(
        _prep_body,
        out_shape=jax.ShapeDtypeStruct((_S ** 6, _NT * _G), jnp.float32),
    )(tab, ws, wm, bg, src2, dst2)


def _make_sc_gather(batch):
    bpw = batch // _NW
    nch = bpw // _CH
    mesh = plsc.VectorSubcoreMesh(core_axis_name="c", subcore_axis_name="s")

    @functools.partial(
        pl.kernel,
        out_type=jax.ShapeDtypeStruct((batch, _NT * _G), jnp.float32),
        mesh=mesh,
        scratch_types=[
            pltpu.VMEM((_NR, bpw), jnp.int32),
            pltpu.VMEM((nch, _CH), jnp.int32),
            pltpu.VMEM((nch, _CH, _NT * _G), jnp.float32),
        ] + [pltpu.SemaphoreType.DMA] * (3 + nch),
    )
    def sc_gather(xt_hbm, tcomb_hbm, out_hbm, xcols, idxs, bufs,
                  sem_x, sem_o0, sem_o1, *sem_g):
        sem_o = (sem_o0, sem_o1)
        wid = lax.axis_index("s") * 2 + lax.axis_index("c")
        base = wid * bpw
        xcps = [pltpu.async_copy(xt_hbm.at[j, pl.ds(base, bpw)], xcols.at[j],
                                 sem_x) for j in range(_NR)]
        for cp in xcps:
            cp.wait()
        for ch in range(nch):
            for v in range(_CH // 16):
                sl = pl.ds(ch * _CH + v * 16, 16)
                dsl = pl.ds(v * 16, 16)
                idxs[ch, dsl] = (xcols[0, sl] * 1024 + xcols[1, sl] * 256
                                 + xcols[2, sl] * 64 + xcols[3, sl] * 16
                                 + xcols[4, sl] * 4 + xcols[5, sl])

        gathers = [pltpu.async_copy(tcomb_hbm.at[idxs.at[ch]], bufs.at[ch],
                                    sem_g[ch]) for ch in range(nch)]
        outs = []
        for ch in range(nch):
            gathers[ch].wait()
            outs.append(pltpu.async_copy(
                bufs.at[ch], out_hbm.at[pl.ds(base + ch * _CH, _CH), :],
                sem_o[ch % 2]))
        for cp in outs:
            cp.wait()

    return sc_gather


def _head_body(t_ref, w1_ref, b1_ref, w2_ref, b2_ref, o_ref):
    z = jnp.dot(t_ref[...].astype(jnp.bfloat16),
                w1_ref[...].astype(jnp.bfloat16),
                preferred_element_type=jnp.float32) + b1_ref[...]
    z = jnp.maximum(z, z * 0.01)
    o_ref[...] = lax.dot_general(
        w2_ref[...].astype(jnp.bfloat16), z.astype(jnp.bfloat16),
        (((0,), (1,)), ((), ())),
        preferred_element_type=jnp.float32) + b2_ref[...]


def _head_t(t, w1, b1, w2, b2t, block):
    batch = t.shape[0]
    fc1 = w1.shape[1]
    ncls = w2.shape[1]
    grid = batch // block
    out_t = pl.pallas_call(
        _head_body,
        grid=(grid,),
        in_specs=[
            pl.BlockSpec((block, t.shape[1]), lambda i: (i, 0)),
            pl.BlockSpec((w1.shape[0], fc1), lambda i: (0, 0)),
            pl.BlockSpec((1, fc1), lambda i: (0, 0)),
            pl.BlockSpec((w1.shape[1], ncls), lambda i: (0, 0)),
            pl.BlockSpec((ncls, 1), lambda i: (0, 0)),
        ],
        out_specs=pl.BlockSpec((ncls, block), lambda i: (0, i)),
        out_shape=jax.ShapeDtypeStruct((ncls, batch), jnp.float32),
    )(t, w1, b1, w2, b2t)
    return out_t


def kernel(X, tables, W_self, W_msg, b_g, W1, b1, W2, b2, src, dst):
    batch = X.shape[0]
    tcomb = _prep(tables.reshape(_N * _S, _D), W_self, W_msg,
                  b_g.reshape(1, -1), src.reshape(1, -1), dst.reshape(1, -1))
    xt = X[:, _LO:_LO + _NR].T
    t = _make_sc_gather(batch)(xt, tcomb)
    return _head_t(t, W1, b1.reshape(1, -1), W2,
                   b2.reshape(-1, 1), 8192).T


# speedup vs baseline: 1.0259x; 1.0259x over previous
import functools

import jax
import jax.numpy as jnp
from jax import lax
from jax.experimental import pallas as pl
from jax.experimental.pallas import tpu as pltpu
from jax.experimental.pallas import tpu_sc as plsc

_N = 32
_S = 4
_D = 32
_G = 32
_T0 = 28
_NT = 4
_LO = 26
_NR = 6
_NW = 32
_CH = 128


def _prep_body(tab_ref, ws_ref, wm_ref, bg_ref, src_ref, dst_ref, out_ref):
    t6 = tab_ref[_LO * _S:(_LO + _NR) * _S, :]
    ps = jnp.dot(t6, ws_ref[...], preferred_element_type=jnp.float32)
    pm = jnp.dot(t6, wm_ref[...], preferred_element_type=jnp.float32)

    e = src_ref.shape[1]
    scol = src_ref[...]
    dcol = dst_ref[...]
    kio = lax.broadcasted_iota(jnp.int32, (_NT, e), 0) + _T0
    ok = (jnp.broadcast_to(dcol, (_NT, e)) == kio).astype(jnp.float32)
    rio = lax.broadcasted_iota(jnp.int32, (_NR * _S, e), 0) // _S + _LO
    sel = (jnp.broadcast_to(scol, (_NR * _S, e)) == rio).astype(jnp.float32)
    cnt = lax.dot_general(sel, ok, (((1,), (1,)), ((), ())),
                          preferred_element_type=jnp.float32)
    deg = lax.dot_general(jnp.ones((1, e), jnp.float32), ok,
                          (((1,), (1,)), ((), ())),
                          preferred_element_type=jnp.float32)
    coeff = cnt / jnp.maximum(deg, 1.0)
    r24 = lax.broadcasted_iota(jnp.int32, (_NR * _S, _NT), 0) // _S + _LO
    k24 = lax.broadcasted_iota(jnp.int32, (_NR * _S, _NT), 1) + _T0
    delta = (r24 == k24).astype(jnp.float32)

    blocks = []
    for k in range(_NT):
        blocks.append(delta[:, k:k + 1] * ps + coeff[:, k:k + 1] * pm)
    m24 = jnp.concatenate(blocks, axis=1)

    ci = lax.broadcasted_iota(jnp.int32, (_S ** 3, _NR * _S), 0)
    ri = lax.broadcasted_iota(jnp.int32, (_S ** 3, _NR * _S), 1)
    node = ri // _S
    st = ri % _S
    sel_a = (((node == 0) & (st == ci // 16))
             | ((node == 1) & (st == (ci // 4) % 4))
             | ((node == 2) & (st == ci % 4))).astype(jnp.float32)
    sel_b = (((node == 3) & (st == ci // 16))
             | ((node == 4) & (st == (ci // 4) % 4))
             | ((node == 5) & (st == ci % 4))).astype(jnp.float32)
    ta = jnp.dot(sel_a, m24, preferred_element_type=jnp.float32)
    tb = jnp.dot(sel_b, m24, preferred_element_type=jnp.float32)
    btile = jnp.concatenate([bg_ref[...]] * _NT, axis=1)
    ta = ta + btile
    fi = lax.broadcasted_iota(jnp.int32, (_S ** 6, _S ** 3), 0)
    fj = lax.broadcasted_iota(jnp.int32, (_S ** 6, _S ** 3), 1)
    rep_a = (fi // (_S ** 3) == fj).astype(jnp.float32)
    rep_b = (fi % (_S ** 3) == fj).astype(jnp.float32)
    full = (jnp.dot(rep_a, ta, preferred_element_type=jnp.float32)
            + jnp.dot(rep_b, tb, preferred_element_type=jnp.float32))
    out_ref[...] = jnp.maximum(full, full * 0.01)


def _prep(tab, ws, wm, bg, src2, dst2):
    return pl.pallas_call(
        _prep_body,
        out_shape=jax.ShapeDtypeStruct((_S ** 6, _NT * _G), jnp.float32),
    )(tab, ws, wm, bg, src2, dst2)


def _make_sc_gather(batch):
    bpw = batch // _NW
    nch = bpw // _CH
    mesh = plsc.VectorSubcoreMesh(core_axis_name="c", subcore_axis_name="s")

    @functools.partial(
        pl.kernel,
        out_type=jax.ShapeDtypeStruct((batch, _NT * _G), jnp.float32),
        mesh=mesh,
        scratch_types=[
            pltpu.VMEM((_NR, bpw), jnp.int32),
            pltpu.VMEM((nch, _CH), jnp.int32),
            pltpu.VMEM((nch, _CH, _NT * _G), jnp.float32),
        ] + [pltpu.SemaphoreType.DMA] * (3 + nch),
    )
    def sc_gather(xt_hbm, tcomb_hbm, out_hbm, xcols, idxs, bufs,
                  sem_x, sem_o0, sem_o1, *sem_g):
        sem_o = (sem_o0, sem_o1)
        wid = lax.axis_index("s") * 2 + lax.axis_index("c")
        base = wid * bpw
        xcps = [pltpu.async_copy(xt_hbm.at[j, pl.ds(base, bpw)], xcols.at[j],
                                 sem_x) for j in range(_NR)]
        for cp in xcps:
            cp.wait()
        for ch in range(nch):
            for v in range(_CH // 16):
                sl = pl.ds(ch * _CH + v * 16, 16)
                dsl = pl.ds(v * 16, 16)
                idxs[ch, dsl] = (xcols[0, sl] * 1024 + xcols[1, sl] * 256
                                 + xcols[2, sl] * 64 + xcols[3, sl] * 16
                                 + xcols[4, sl] * 4 + xcols[5, sl])

        gathers = [pltpu.async_copy(tcomb_hbm.at[idxs.at[ch]], bufs.at[ch],
                                    sem_g[ch]) for ch in range(nch)]
        outs = []
        for ch in range(nch):
            gathers[ch].wait()
            outs.append(pltpu.async_copy(
                bufs.at[ch], out_hbm.at[pl.ds(base + ch * _CH, _CH), :],
                sem_o[ch % 2]))
        for cp in outs:
            cp.wait()

    return sc_gather


def _head_body(t_ref, w1_ref, b1_ref, w2_ref, b2_ref, o_ref):
    z = jnp.dot(t_ref[...].astype(jnp.bfloat16),
                w1_ref[...].astype(jnp.bfloat16),
                preferred_element_type=jnp.float32) + b1_ref[...]
    z = jnp.maximum(z, z * 0.01)
    o_ref[...] = lax.dot_general(
        w2_ref[...].astype(jnp.bfloat16), z.astype(jnp.bfloat16),
        (((0,), (1,)), ((), ())),
        preferred_element_type=jnp.float32) + b2_ref[...]


def _head_t(t, w1, b1, w2, b2t, block):
    batch = t.shape[0]
    fc1 = w1.shape[1]
    ncls = w2.shape[1]
    grid = batch // block
    out_t = pl.pallas_call(
        _head_body,
        grid=(grid,),
        in_specs=[
            pl.BlockSpec((block, t.shape[1]), lambda i: (i, 0)),
            pl.BlockSpec((w1.shape[0], fc1), lambda i: (0, 0)),
            pl.BlockSpec((1, fc1), lambda i: (0, 0)),
            pl.BlockSpec((w1.shape[1], ncls), lambda i: (0, 0)),
            pl.BlockSpec((ncls, 1), lambda i: (0, 0)),
        ],
        out_specs=pl.BlockSpec((ncls, block), lambda i: (0, i)),
        out_shape=jax.ShapeDtypeStruct((ncls, batch), jnp.float32),
    )(t, w1, b1, w2, b2t)
    return out_t


def kernel(X, tables, W_self, W_msg, b_g, W1, b1, W2, b2, src, dst):
    batch = X.shape[0]
    tcomb = _prep(tables.reshape(_N * _S, _D), W_self, W_msg,
                  b_g.reshape(1, -1), src.reshape(1, -1), dst.reshape(1, -1))
    xt = X[:, _LO:_LO + _NR].T
    t = _make_sc_gather(batch)(xt, tcomb)
    return _head_t(t, W1, b1.reshape(1, -1), W2,
                   b2.reshape(-1, 1), 8192).T
